# baseline (speedup 1.0000x reference)
import functools
import math

import jax
import jax.numpy as jnp
from jax import lax
from jax.experimental import pallas as pl
from jax.experimental.pallas import tpu as pltpu
from jax.experimental.pallas import tpu_sc as plsc

_NC = 80
_NA = 3
_ANCHOR_T = 4.0
_ANCHORS = (
    ((1.25, 1.625), (2.0, 3.75), (4.125, 2.875)),
    ((1.875, 3.8125), (3.875, 2.8125), (3.6875, 7.4375)),
    ((3.625, 2.8125), (4.875, 6.1875), (11.65625, 10.1875)),
)
_BAL = (4.0, 1.0, 0.4)
_P = 3072
_NW = 32
_PW = _P // _NW
_DB = 4096


def _softplus(x):
    return jnp.maximum(x, 0.0) + jnp.log1p(jnp.exp(-jnp.abs(x)))


def _atan_pos(x):
    t1 = x > 2.414213562373095
    t2 = x > 0.4142135623730951
    xr = jnp.where(t1, -1.0 / jnp.maximum(x, 1e-30),
                   jnp.where(t2, (x - 1.0) / (x + 1.0), x))
    y0 = jnp.where(t1, math.pi / 2, jnp.where(t2, math.pi / 4, 0.0))
    z = xr * xr
    p = (((8.05374449538e-2 * z - 1.38776856032e-1) * z
          + 1.99777106478e-1) * z - 3.33329491539e-1) * z * xr + xr
    return y0 + p


_CH = 24


def _sc_gather(t0, t1, t2, i0, i1, i2):
    mesh = plsc.VectorSubcoreMesh(core_axis_name="c", subcore_axis_name="s")

    @functools.partial(
        pl.kernel,
        out_type=tuple(jax.ShapeDtypeStruct((_P, 85), jnp.float32) for _ in range(3)),
        mesh=mesh,
        scratch_types=(
            pltpu.VMEM((_PW,), jnp.int32),
            pltpu.VMEM((_PW, 85), jnp.float32),
            pltpu.SemaphoreType.DMA,
        ),
        compiler_params=pltpu.CompilerParams(needs_layout_passes=False),
    )
    def gather_k(t0r, t1r, t2r, i0r, i1r, i2r, o0r, o1r, o2r,
                 idx_v, rows, sem):
        wid = lax.axis_index("s") * 2 + lax.axis_index("c")
        base = wid * _PW
        tabs = (t0r, t1r, t2r)
        idxs = (i0r, i1r, i2r)
        outs = (o0r, o1r, o2r)
        lanes = lax.iota(jnp.int32, 16)
        for l in range(3):
            pltpu.sync_copy(idxs[l].at[pl.ds(base, _PW)], idx_v)
            for g in range(_PW // 16):
                vec = idx_v[pl.ds(g * 16, 16)]

                def issue(c, _, l=l, g=g, vec=vec):
                    k = jnp.sum(jnp.where(lanes == c, vec, 0))
                    pltpu.async_copy(
                        tabs[l].at[pl.ds(k, 1)],
                        rows.at[pl.ds(g * 16 + c, 1)], sem)
                    return 0

                lax.fori_loop(0, 16, issue, 0)
                pltpu.make_async_copy(
                    tabs[l].at[pl.ds(0, 16)],
                    rows.at[pl.ds(g * 16, 16)], sem).wait()
            pltpu.sync_copy(rows, outs[l].at[pl.ds(base, _PW)])

    return gather_k(t0, t1, t2, i0, i1, i2)


def _dense_obj_sums(f0, f1, f2):
    nblocks = f0.shape[0] // _DB

    def body(x0_ref, x1_ref, x2_ref, o_ref):
        i = pl.program_id(0)

        @pl.when(i == 0)
        def _():
            o_ref[...] = jnp.zeros_like(o_ref)

        r = lax.broadcasted_iota(jnp.int32, (8, 128), 0)
        c = lax.broadcasted_iota(jnp.int32, (8, 128), 1)

        def accum(x_ref, lvl):
            x = x_ref[:, 4:5].reshape(_DB // 128, 128)
            s = jnp.sum(_softplus(x))
            o_ref[...] += jnp.where((r == lvl) & (c == 0), s, 0.0)

        accum(x0_ref, 0)

        @pl.when(i % 4 == 0)
        def _():
            accum(x1_ref, 1)

        @pl.when(i % 16 == 0)
        def _():
            accum(x2_ref, 2)

    return pl.pallas_call(
        body,
        grid=(nblocks,),
        in_specs=[
            pl.BlockSpec((_DB, 85), lambda i: (i, 0)),
            pl.BlockSpec((_DB, 85), lambda i: (i // 4, 0)),
            pl.BlockSpec((_DB, 85), lambda i: (i // 16, 0)),
        ],
        out_specs=pl.BlockSpec((8, 128), lambda i: (0, 0)),
        out_shape=jax.ShapeDtypeStruct((8, 128), jnp.float32),
        compiler_params=pltpu.CompilerParams(
            dimension_semantics=("arbitrary",)),
    )(f0, f1, f2)


def _cand_stats(psT, meta):
    def body(ps_ref, mt_ref, o_ref):
        eps = 1e-7
        rows = []
        for l in range(3):
            r0 = 24 * l

            def mt(c):
                return mt_ref[c, r0:r0 + 24, :]

            def ps(c):
                return ps_ref[c, r0:r0 + 24, :]

            mf = mt(6)
            sig0 = 1.0 / (1.0 + jnp.exp(-ps(0)))
            sig1 = 1.0 / (1.0 + jnp.exp(-ps(1)))
            sig2 = 1.0 / (1.0 + jnp.exp(-ps(2)))
            sig3 = 1.0 / (1.0 + jnp.exp(-ps(3)))
            pxw = sig0 * 2.0 - 0.5
            pyw = sig1 * 2.0 - 0.5
            pww = (sig2 * 2.0) ** 2 * mt(4)
            phw = (sig3 * 2.0) ** 2 * mt(5)
            b1x1 = pxw - pww / 2
            b1x2 = pxw + pww / 2
            b1y1 = pyw - phw / 2
            b1y2 = pyw + phw / 2
            b2x1 = mt(0) - mt(2) / 2
            b2x2 = mt(0) + mt(2) / 2
            b2y1 = mt(1) - mt(3) / 2
            b2y2 = mt(1) + mt(3) / 2
            inter = (jnp.clip(jnp.minimum(b1x2, b2x2) - jnp.maximum(b1x1, b2x1), 0.0, None)
                     * jnp.clip(jnp.minimum(b1y2, b2y2) - jnp.maximum(b1y1, b2y1), 0.0, None))
            w1 = b1x2 - b1x1
            h1 = b1y2 - b1y1 + eps
            w2 = b2x2 - b2x1
            h2 = b2y2 - b2y1 + eps
            union = w1 * h1 + w2 * h2 - inter + eps
            iou = inter / union
            cw = jnp.maximum(b1x2, b2x2) - jnp.minimum(b1x1, b2x1)
            ch = jnp.maximum(b1y2, b2y2) - jnp.minimum(b1y1, b2y1)
            c2 = cw ** 2 + ch ** 2 + eps
            rho2 = ((b2x1 + b2x2 - b1x1 - b1x2) ** 2
                    + (b2y1 + b2y2 - b1y1 - b1y2) ** 2) / 4.0
            v = (4.0 / math.pi ** 2) * (_atan_pos(w2 / h2) - _atan_pos(w1 / h1)) ** 2
            alpha = v / (v - iou + (1.0 + eps))
            ciou = iou - (rho2 / c2 + v * alpha)

            cnt = jnp.maximum(jnp.sum(mf), 1.0)
            lbox = jnp.sum((1.0 - ciou) * mf)
            xval = jnp.sum(ps(4) * jnp.clip(iou, 0.0, None) * mf)

            clsid = mt(7)
            spsum = jnp.zeros((24, 128), jnp.float32)
            clslogit = jnp.zeros((24, 128), jnp.float32)
            for c in range(_NC):
                x = ps(5 + c)
                spsum += _softplus(x)
                clslogit += jnp.where(clsid == float(c), x, 0.0)
            lcls = jnp.sum((spsum - clslogit) * mf)

            lane = lax.broadcasted_iota(jnp.int32, (1, 128), 1)
            row = jnp.where(lane == 0, cnt,
                            jnp.where(lane == 1, lbox,
                                      jnp.where(lane == 2, lcls,
                                                jnp.where(lane == 3, xval, 0.0))))
            rows.append(row)
        o_ref[...] = jnp.concatenate(rows + [jnp.zeros((5, 128), jnp.float32)], axis=0)

    return pl.pallas_call(
        body,
        out_shape=jax.ShapeDtypeStruct((8, 128), jnp.float32),
    )(psT, meta)


def _build_candidates(targets, shapes):
    nt = targets.shape[0]
    ai = jnp.tile(jnp.arange(_NA, dtype=jnp.float32)[:, None], (1, nt))
    t_all = jnp.concatenate(
        [jnp.tile(targets[None], (_NA, 1, 1)), ai[..., None]], axis=2)
    g = 0.5
    off = jnp.array([[0, 0], [1, 0], [0, 1], [-1, 0], [0, -1]],
                    dtype=jnp.float32) * g
    out = []
    for i in range(3):
        anchors = jnp.array(_ANCHORS[i], dtype=jnp.float32)
        H, W = shapes[i][2], shapes[i][3]
        gain = jnp.array([1, 1, W, H, W, H, 1], dtype=jnp.float32)
        t = t_all * gain
        r = t[..., 4:6] / anchors[:, None, :]
        j = jnp.max(jnp.maximum(r, 1.0 / r), axis=-1) < _ANCHOR_T
        t_f = t.reshape(_NA * nt, 7)
        j_f = j.reshape(_NA * nt)
        gxy = t_f[:, 2:4]
        gxi = gain[2:4] - gxy
        jj, kk = ((gxy % 1.0 < g) & (gxy > 1.0)).T
        ll, mm = ((gxi % 1.0 < g) & (gxi > 1.0)).T
        jmask = jnp.stack([jnp.ones_like(jj), jj, kk, ll, mm]) & j_f[None]
        n5 = 5 * _NA * nt
        t_rep = jnp.tile(t_f[None], (5, 1, 1)).reshape(n5, 7)
        offsets = (jnp.zeros_like(gxy)[None] + off[:, None, :]).reshape(n5, 2)
        m_rep = jmask.reshape(n5)
        b = t_rep[:, 0].astype(jnp.int32)
        c = t_rep[:, 1]
        gxy = t_rep[:, 2:4]
        gwh = t_rep[:, 4:6]
        gij = (gxy - offsets).astype(jnp.int32)
        gi = jnp.clip(gij[:, 0], 0, W - 1)
        gj = jnp.clip(gij[:, 1], 0, H - 1)
        a = t_rep[:, 6].astype(jnp.int32)
        lin = ((b * _NA + a) * H + gj) * W + gi
        tb = jnp.concatenate([gxy - gij.astype(jnp.float32), gwh], axis=1)
        anch = anchors[a]
        pad = _P - n5

        def padded(x):
            return jnp.pad(x, ((0, pad),) + ((0, 0),) * (x.ndim - 1))

        out.append(dict(
            idx=padded(lin),
            tbox=padded(tb),
            anch=padded(anch),
            cls=padded(c),
            mask=padded(m_rep.astype(jnp.float32)),
        ))
    return out


def kernel(pred0, pred1, pred2, targets):
    preds = (pred0, pred1, pred2)
    shapes = [p.shape for p in preds]
    flats = [p.reshape(-1, 85) for p in preds]
    cands = _build_candidates(targets, shapes)

    ps0, ps1, ps2 = _sc_gather(
        flats[0], flats[1], flats[2],
        cands[0]["idx"], cands[1]["idx"], cands[2]["idx"])

    dsums = _dense_obj_sums(flats[0], flats[1], flats[2])
    dense = [dsums[l, 0] for l in range(3)]

    ps_all = jnp.concatenate([ps0, ps1, ps2], axis=0)
    psT = ps_all.T.reshape(85, 72, 128)
    meta = jnp.stack([
        jnp.concatenate([c["tbox"][:, 0] for c in cands]),
        jnp.concatenate([c["tbox"][:, 1] for c in cands]),
        jnp.concatenate([c["tbox"][:, 2] for c in cands]),
        jnp.concatenate([c["tbox"][:, 3] for c in cands]),
        jnp.concatenate([c["anch"][:, 0] for c in cands]),
        jnp.concatenate([c["anch"][:, 1] for c in cands]),
        jnp.concatenate([c["mask"] for c in cands]),
        jnp.concatenate([c["cls"] for c in cands]),
    ]).reshape(8, 72, 128)

    stats = _cand_stats(psT, meta)

    loss = jnp.float32(0.0)
    for l in range(3):
        cnt = stats[l, 0]
        lbox = stats[l, 1]
        lcls = stats[l, 2]
        xval = stats[l, 3]
        n_l = flats[l].shape[0]
        lobj = (dense[l] - xval) / n_l * _BAL[l]
        loss = loss + lbox / cnt + 0.5 * lobj + 0.05 * lcls / (cnt * _NC)
    return loss


# speedup vs baseline: 1.0136x; 1.0136x over previous
import functools
import math

import jax
import jax.numpy as jnp
from jax import lax
from jax.experimental import pallas as pl
from jax.experimental.pallas import tpu as pltpu
from jax.experimental.pallas import tpu_sc as plsc

_NC = 80
_NA = 3
_ANCHOR_T = 4.0
_ANCHORS = (
    ((1.25, 1.625), (2.0, 3.75), (4.125, 2.875)),
    ((1.875, 3.8125), (3.875, 2.8125), (3.6875, 7.4375)),
    ((3.625, 2.8125), (4.875, 6.1875), (11.65625, 10.1875)),
)
_BAL = (4.0, 1.0, 0.4)
_P = 3072
_NW = 32
_PW = _P // _NW
_DB = 4096


def _softplus(x):
    return jnp.maximum(x, 0.0) + jnp.log1p(jnp.exp(-jnp.abs(x)))


def _atan_pos(x):
    t1 = x > 2.414213562373095
    t2 = x > 0.4142135623730951
    xr = jnp.where(t1, -1.0 / jnp.maximum(x, 1e-30),
                   jnp.where(t2, (x - 1.0) / (x + 1.0), x))
    y0 = jnp.where(t1, math.pi / 2, jnp.where(t2, math.pi / 4, 0.0))
    z = xr * xr
    p = (((8.05374449538e-2 * z - 1.38776856032e-1) * z
          + 1.99777106478e-1) * z - 3.33329491539e-1) * z * xr + xr
    return y0 + p


_CH = 24


def _sc_gather(t0, t1, t2, i0, i1, i2):
    mesh = plsc.VectorSubcoreMesh(core_axis_name="c", subcore_axis_name="s")

    @functools.partial(
        pl.kernel,
        out_type=tuple(jax.ShapeDtypeStruct((_P, 85), jnp.float32) for _ in range(3)),
        mesh=mesh,
        scratch_types=(
            pltpu.VMEM((_PW,), jnp.int32),
            pltpu.VMEM((_PW, 85), jnp.float32),
            pltpu.SemaphoreType.DMA,
        ),
        compiler_params=pltpu.CompilerParams(needs_layout_passes=False),
    )
    def gather_k(t0r, t1r, t2r, i0r, i1r, i2r, o0r, o1r, o2r,
                 idx_v, rows, sem):
        wid = lax.axis_index("s") * 2 + lax.axis_index("c")
        base = wid * _PW
        tabs = (t0r, t1r, t2r)
        idxs = (i0r, i1r, i2r)
        outs = (o0r, o1r, o2r)
        lanes = lax.iota(jnp.int32, 16)
        for l in range(3):
            pltpu.sync_copy(idxs[l].at[pl.ds(base, _PW)], idx_v)
            for g in range(_PW // 16):
                vec = idx_v[pl.ds(g * 16, 16)]

                def issue(c, _, l=l, g=g, vec=vec):
                    k = jnp.sum(jnp.where(lanes == c, vec, 0))
                    pltpu.async_copy(
                        tabs[l].at[pl.ds(k, 1)],
                        rows.at[pl.ds(g * 16 + c, 1)], sem)
                    return 0

                lax.fori_loop(0, 16, issue, 0)
                pltpu.make_async_copy(
                    tabs[l].at[pl.ds(0, 16)],
                    rows.at[pl.ds(g * 16, 16)], sem).wait()
            pltpu.sync_copy(rows, outs[l].at[pl.ds(base, _PW)])

    return gather_k(t0, t1, t2, i0, i1, i2)


def _dense_obj_sums(f0, f1, f2):
    nblocks = f0.shape[0] // _DB

    def body(x0_ref, x1_ref, x2_ref, o_ref):
        i = pl.program_id(0)

        @pl.when(i == 0)
        def _():
            o_ref[...] = jnp.zeros_like(o_ref)

        r = lax.broadcasted_iota(jnp.int32, (8, 128), 0)
        c = lax.broadcasted_iota(jnp.int32, (8, 128), 1)

        def accum(x_ref, lvl):
            x = x_ref[:, 4:5].reshape(_DB // 128, 128)
            s = jnp.sum(_softplus(x))
            o_ref[...] += jnp.where((r == lvl) & (c == 0), s, 0.0)

        accum(x0_ref, 0)

        @pl.when(i % 4 == 0)
        def _():
            accum(x1_ref, 1)

        @pl.when(i % 16 == 0)
        def _():
            accum(x2_ref, 2)

    return pl.pallas_call(
        body,
        grid=(nblocks,),
        in_specs=[
            pl.BlockSpec((_DB, 85), lambda i: (i, 0)),
            pl.BlockSpec((_DB, 85), lambda i: (i // 4, 0)),
            pl.BlockSpec((_DB, 85), lambda i: (i // 16, 0)),
        ],
        out_specs=pl.BlockSpec((8, 128), lambda i: (0, 0)),
        out_shape=jax.ShapeDtypeStruct((8, 128), jnp.float32),
        compiler_params=pltpu.CompilerParams(
            dimension_semantics=("arbitrary",)),
    )(f0, f1, f2)


def _cand_stats(psT, meta):
    def body(ps_ref, mt_ref, o_ref):
        eps = 1e-7
        cls_full = mt_ref[7]
        acc_sp = [jnp.zeros((72, 128), jnp.float32) for _ in range(4)]
        acc_cl = [jnp.zeros((72, 128), jnp.float32) for _ in range(4)]
        for c in range(_NC):
            x = ps_ref[5 + c]
            acc_sp[c % 4] = acc_sp[c % 4] + _softplus(x)
            acc_cl[c % 4] = acc_cl[c % 4] + jnp.where(cls_full == float(c), x, 0.0)
        lcls_full = ((acc_sp[0] + acc_sp[1]) + (acc_sp[2] + acc_sp[3])
                     - ((acc_cl[0] + acc_cl[1]) + (acc_cl[2] + acc_cl[3])))

        rows = []
        for l in range(3):
            r0 = 24 * l

            def mt(c):
                return mt_ref[c, r0:r0 + 24, :]

            def ps(c):
                return ps_ref[c, r0:r0 + 24, :]

            mf = mt(6)
            sig0 = 1.0 / (1.0 + jnp.exp(-ps(0)))
            sig1 = 1.0 / (1.0 + jnp.exp(-ps(1)))
            sig2 = 1.0 / (1.0 + jnp.exp(-ps(2)))
            sig3 = 1.0 / (1.0 + jnp.exp(-ps(3)))
            pxw = sig0 * 2.0 - 0.5
            pyw = sig1 * 2.0 - 0.5
            pww = (sig2 * 2.0) ** 2 * mt(4)
            phw = (sig3 * 2.0) ** 2 * mt(5)
            b1x1 = pxw - pww / 2
            b1x2 = pxw + pww / 2
            b1y1 = pyw - phw / 2
            b1y2 = pyw + phw / 2
            b2x1 = mt(0) - mt(2) / 2
            b2x2 = mt(0) + mt(2) / 2
            b2y1 = mt(1) - mt(3) / 2
            b2y2 = mt(1) + mt(3) / 2
            inter = (jnp.clip(jnp.minimum(b1x2, b2x2) - jnp.maximum(b1x1, b2x1), 0.0, None)
                     * jnp.clip(jnp.minimum(b1y2, b2y2) - jnp.maximum(b1y1, b2y1), 0.0, None))
            w1 = b1x2 - b1x1
            h1 = b1y2 - b1y1 + eps
            w2 = b2x2 - b2x1
            h2 = b2y2 - b2y1 + eps
            union = w1 * h1 + w2 * h2 - inter + eps
            iou = inter / union
            cw = jnp.maximum(b1x2, b2x2) - jnp.minimum(b1x1, b2x1)
            ch = jnp.maximum(b1y2, b2y2) - jnp.minimum(b1y1, b2y1)
            c2 = cw ** 2 + ch ** 2 + eps
            rho2 = ((b2x1 + b2x2 - b1x1 - b1x2) ** 2
                    + (b2y1 + b2y2 - b1y1 - b1y2) ** 2) / 4.0
            v = (4.0 / math.pi ** 2) * (_atan_pos(w2 / h2) - _atan_pos(w1 / h1)) ** 2
            alpha = v / (v - iou + (1.0 + eps))
            ciou = iou - (rho2 / c2 + v * alpha)

            rows.append(jnp.sum(mf, axis=0, keepdims=True))
            rows.append(jnp.sum((1.0 - ciou) * mf, axis=0, keepdims=True))
            rows.append(jnp.sum(lcls_full[r0:r0 + 24, :] * mf,
                                axis=0, keepdims=True))
            rows.append(jnp.sum(ps(4) * jnp.clip(iou, 0.0, None) * mf,
                                axis=0, keepdims=True))
        o_ref[...] = jnp.concatenate(rows + [jnp.zeros((4, 128), jnp.float32)],
                                     axis=0)

    return pl.pallas_call(
        body,
        out_shape=jax.ShapeDtypeStruct((16, 128), jnp.float32),
    )(psT, meta)


def _build_candidates(targets, shapes):
    nt = targets.shape[0]
    ai = jnp.tile(jnp.arange(_NA, dtype=jnp.float32)[:, None], (1, nt))
    t_all = jnp.concatenate(
        [jnp.tile(targets[None], (_NA, 1, 1)), ai[..., None]], axis=2)
    g = 0.5
    off = jnp.array([[0, 0], [1, 0], [0, 1], [-1, 0], [0, -1]],
                    dtype=jnp.float32) * g
    out = []
    for i in range(3):
        anchors = jnp.array(_ANCHORS[i], dtype=jnp.float32)
        H, W = shapes[i][2], shapes[i][3]
        gain = jnp.array([1, 1, W, H, W, H, 1], dtype=jnp.float32)
        t = t_all * gain
        r = t[..., 4:6] / anchors[:, None, :]
        j = jnp.max(jnp.maximum(r, 1.0 / r), axis=-1) < _ANCHOR_T
        t_f = t.reshape(_NA * nt, 7)
        j_f = j.reshape(_NA * nt)
        gxy = t_f[:, 2:4]
        gxi = gain[2:4] - gxy
        jj, kk = ((gxy % 1.0 < g) & (gxy > 1.0)).T
        ll, mm = ((gxi % 1.0 < g) & (gxi > 1.0)).T
        jmask = jnp.stack([jnp.ones_like(jj), jj, kk, ll, mm]) & j_f[None]
        n5 = 5 * _NA * nt
        t_rep = jnp.tile(t_f[None], (5, 1, 1)).reshape(n5, 7)
        offsets = (jnp.zeros_like(gxy)[None] + off[:, None, :]).reshape(n5, 2)
        m_rep = jmask.reshape(n5)
        b = t_rep[:, 0].astype(jnp.int32)
        c = t_rep[:, 1]
        gxy = t_rep[:, 2:4]
        gwh = t_rep[:, 4:6]
        gij = (gxy - offsets).astype(jnp.int32)
        gi = jnp.clip(gij[:, 0], 0, W - 1)
        gj = jnp.clip(gij[:, 1], 0, H - 1)
        a = t_rep[:, 6].astype(jnp.int32)
        lin = ((b * _NA + a) * H + gj) * W + gi
        tb = jnp.concatenate([gxy - gij.astype(jnp.float32), gwh], axis=1)
        anch = anchors[a]
        pad = _P - n5

        def padded(x):
            return jnp.pad(x, ((0, pad),) + ((0, 0),) * (x.ndim - 1))

        out.append(dict(
            idx=padded(lin),
            tbox=padded(tb),
            anch=padded(anch),
            cls=padded(c),
            mask=padded(m_rep.astype(jnp.float32)),
        ))
    return out


def kernel(pred0, pred1, pred2, targets):
    preds = (pred0, pred1, pred2)
    shapes = [p.shape for p in preds]
    flats = [p.reshape(-1, 85) for p in preds]
    cands = _build_candidates(targets, shapes)

    ps0, ps1, ps2 = _sc_gather(
        flats[0], flats[1], flats[2],
        cands[0]["idx"], cands[1]["idx"], cands[2]["idx"])

    dsums = _dense_obj_sums(flats[0], flats[1], flats[2])
    dense = [dsums[l, 0] for l in range(3)]

    ps_all = jnp.concatenate([ps0, ps1, ps2], axis=0)
    psT = ps_all.T.reshape(85, 72, 128)
    meta = jnp.stack([
        jnp.concatenate([c["tbox"][:, 0] for c in cands]),
        jnp.concatenate([c["tbox"][:, 1] for c in cands]),
        jnp.concatenate([c["tbox"][:, 2] for c in cands]),
        jnp.concatenate([c["tbox"][:, 3] for c in cands]),
        jnp.concatenate([c["anch"][:, 0] for c in cands]),
        jnp.concatenate([c["anch"][:, 1] for c in cands]),
        jnp.concatenate([c["mask"] for c in cands]),
        jnp.concatenate([c["cls"] for c in cands]),
    ]).reshape(8, 72, 128)

    stats = jnp.sum(_cand_stats(psT, meta), axis=1)

    loss = jnp.float32(0.0)
    for l in range(3):
        cnt = jnp.maximum(stats[4 * l + 0], 1.0)
        lbox = stats[4 * l + 1]
        lcls = stats[4 * l + 2]
        xval = stats[4 * l + 3]
        n_l = flats[l].shape[0]
        lobj = (dense[l] - xval) / n_l * _BAL[l]
        loss = loss + lbox / cnt + 0.5 * lobj + 0.05 * lcls / (cnt * _NC)
    return loss
